# baseline (speedup 1.0000x reference)
import functools

import jax
import jax.numpy as jnp
from jax import lax
from jax.experimental import pallas as pl
from jax.experimental.pallas import tpu as pltpu

_NEG = -1e30


def _round_up(x, m):
    return ((x + m - 1) // m) * m


def _gather_ce_kernel(idx_sref, tgt_ref, emb_ref, logits_ref, part_ref,
                      *, tile, groups_per_trip, v_pad, bt):
    i = pl.program_id(0)
    base = i * tile

    def trip(o, carry):
        for g in range(groups_per_trip):
            row0 = o * groups_per_trip * 8 + g * 8
            rows = [emb_ref[idx_sref[base + row0 + k], 0, :]
                    for k in range(8)]
            x8 = jnp.stack(rows, axis=0)
            logits_ref[pl.ds(pl.multiple_of(row0, 8), 8), :] = x8
        return carry

    lax.fori_loop(0, tile // (8 * groups_per_trip), trip, 0)

    x = logits_ref[...]
    m = jnp.max(x, axis=-1, keepdims=True)
    s = jnp.sum(jnp.exp(x - m), axis=-1, keepdims=True)
    lane = lax.broadcasted_iota(jnp.int32, x.shape, 1)
    tgt_logit = jnp.sum(jnp.where(lane == tgt_ref[...], x, 0.0),
                        axis=-1, keepdims=True)
    per_ex = m + jnp.log(s) - tgt_logit
    row_id = base + lax.broadcasted_iota(jnp.int32, (tile, 1), 0)
    per_ex = jnp.where(row_id < bt, per_ex, 0.0)
    part_ref[0, 0, :] = jnp.sum(per_ex).reshape(1)


def _pad_1d(tok, bt, bt_pad):
    tok = tok.reshape(bt).astype(jnp.int32)
    if bt_pad != bt:
        tok = jnp.concatenate([tok, jnp.zeros((bt_pad - bt,), jnp.int32)])
    return tok


def kernel(emb_table, idx, targets, *, tile=2048, groups_per_trip=4):
    B, T = idx.shape
    V = emb_table.shape[0]
    BT = B * T

    v_pad = _round_up(V, 128)
    tile = min(tile, _round_up(BT, 8))
    bt_pad = _round_up(BT, tile)
    num_tiles = bt_pad // tile

    if v_pad == V:
        emb_pad = emb_table.astype(jnp.float32)
    else:
        emb_pad = jnp.full((v_pad, v_pad), _NEG, dtype=jnp.float32)
        emb_pad = emb_pad.at[:V, :V].set(emb_table.astype(jnp.float32))

    emb3 = emb_pad.reshape(v_pad, 1, v_pad)
    idx_flat = _pad_1d(idx, BT, bt_pad)
    tgt_flat = _pad_1d(targets if targets is not None else idx, BT, bt_pad)
    tgt2 = tgt_flat.reshape(bt_pad, 1)

    body = functools.partial(_gather_ce_kernel, tile=tile,
                             groups_per_trip=groups_per_trip,
                             v_pad=v_pad, bt=BT)

    logits, partials = pl.pallas_call(
        body,
        out_shape=(
            jax.ShapeDtypeStruct((bt_pad, v_pad), jnp.float32),
            jax.ShapeDtypeStruct((num_tiles, 1, 1), jnp.float32),
        ),
        grid=(num_tiles,),
        in_specs=[
            pl.BlockSpec(memory_space=pltpu.SMEM),
            pl.BlockSpec((tile, 1), lambda i: (i, 0)),
            pl.BlockSpec((v_pad, 1, v_pad), lambda i: (0, 0, 0)),
        ],
        out_specs=(
            pl.BlockSpec((tile, v_pad), lambda i: (i, 0)),
            pl.BlockSpec((1, 1, 1), lambda i: (i, 0, 0)),
        ),
        compiler_params=pltpu.CompilerParams(
            dimension_semantics=("parallel",),
            vmem_limit_bytes=58 * 1024 * 1024,
        ),
        cost_estimate=pl.CostEstimate(
            flops=8 * bt_pad * v_pad,
            transcendentals=bt_pad * v_pad,
            bytes_accessed=(v_pad * v_pad * 4 + bt_pad * v_pad * 4
                            + 2 * bt_pad * 4),
        ),
    )(idx_flat, tgt2, emb3)

    if bt_pad != BT or v_pad != V:
        logits = logits[:BT, :V]

    if targets is None:
        return logits.reshape(B, T, V), None

    return logits, jnp.sum(partials) / BT


# speedup vs baseline: 1.0446x; 1.0446x over previous
import functools

import jax
import jax.numpy as jnp
from jax import lax
from jax.experimental import pallas as pl
from jax.experimental.pallas import tpu as pltpu

_NEG = -1e30


def _round_up(x, m):
    return ((x + m - 1) // m) * m


def _gather_ce_kernel(idx_sref, tgt_ref, emb_ref, logits_ref, part_ref,
                      emb3_scr, copy_sem,
                      *, tile, groups_per_trip, v_pad, bt):
    i = pl.program_id(0)
    base = i * tile

    @pl.when(i == 0)
    def _():
        cp = pltpu.make_async_copy(emb_ref, emb3_scr.at[:, 0, :], copy_sem)
        cp.start()
        cp.wait()

    def trip(o, carry):
        for g in range(groups_per_trip):
            row0 = o * groups_per_trip * 8 + g * 8
            rows = [emb3_scr[idx_sref[base + row0 + k], 0, :]
                    for k in range(8)]
            x8 = jnp.stack(rows, axis=0)
            logits_ref[pl.ds(pl.multiple_of(row0, 8), 8), :] = x8
        return carry

    lax.fori_loop(0, tile // (8 * groups_per_trip), trip, 0)

    x = logits_ref[...]
    m = jnp.max(x, axis=-1, keepdims=True)
    s = jnp.sum(jnp.exp(x - m), axis=-1, keepdims=True)
    lane = lax.broadcasted_iota(jnp.int32, x.shape, 1)
    tgt_logit = jnp.sum(jnp.where(lane == tgt_ref[...], x, 0.0),
                        axis=-1, keepdims=True)
    per_ex = m + jnp.log(s) - tgt_logit
    row_id = base + lax.broadcasted_iota(jnp.int32, (tile, 1), 0)
    per_ex = jnp.where(row_id < bt, per_ex, 0.0)
    part_ref[0, 0, :] = jnp.sum(per_ex).reshape(1)


def _pad_1d(tok, bt, bt_pad):
    tok = tok.reshape(bt).astype(jnp.int32)
    if bt_pad != bt:
        tok = jnp.concatenate([tok, jnp.zeros((bt_pad - bt,), jnp.int32)])
    return tok


def kernel(emb_table, idx, targets, *, tile=1024, groups_per_trip=4):
    B, T = idx.shape
    V = emb_table.shape[0]
    BT = B * T

    v_pad = _round_up(V, 128)
    tile = min(tile, _round_up(BT, 8))
    bt_pad = _round_up(BT, tile)
    num_tiles = bt_pad // tile

    if v_pad == V:
        emb_pad = emb_table.astype(jnp.float32)
    else:
        emb_pad = jnp.full((v_pad, v_pad), _NEG, dtype=jnp.float32)
        emb_pad = emb_pad.at[:V, :V].set(emb_table.astype(jnp.float32))

    idx_flat = _pad_1d(idx, BT, bt_pad)
    tgt_flat = _pad_1d(targets if targets is not None else idx, BT, bt_pad)
    tgt2 = tgt_flat.reshape(bt_pad, 1)

    body = functools.partial(_gather_ce_kernel, tile=tile,
                             groups_per_trip=groups_per_trip,
                             v_pad=v_pad, bt=BT)

    logits, partials = pl.pallas_call(
        body,
        out_shape=(
            jax.ShapeDtypeStruct((bt_pad, v_pad), jnp.float32),
            jax.ShapeDtypeStruct((num_tiles, 1, 1), jnp.float32),
        ),
        grid=(num_tiles,),
        in_specs=[
            pl.BlockSpec(memory_space=pltpu.SMEM),
            pl.BlockSpec((tile, 1), lambda i: (i, 0)),
            pl.BlockSpec((v_pad, v_pad), lambda i: (0, 0)),
        ],
        out_specs=(
            pl.BlockSpec((tile, v_pad), lambda i: (i, 0)),
            pl.BlockSpec((1, 1, 1), lambda i: (i, 0, 0)),
        ),
        scratch_shapes=[
            pltpu.VMEM((v_pad, 1, v_pad), jnp.float32),
            pltpu.SemaphoreType.DMA,
        ],
        compiler_params=pltpu.CompilerParams(
            dimension_semantics=("arbitrary",),
            vmem_limit_bytes=58 * 1024 * 1024,
        ),
        cost_estimate=pl.CostEstimate(
            flops=8 * bt_pad * v_pad,
            transcendentals=bt_pad * v_pad,
            bytes_accessed=(v_pad * v_pad * 4 + bt_pad * v_pad * 4
                            + 2 * bt_pad * 4),
        ),
    )(idx_flat, tgt2, emb_pad)

    if bt_pad != BT or v_pad != V:
        logits = logits[:BT, :V]

    if targets is None:
        return logits.reshape(B, T, V), None

    return logits, jnp.sum(partials) / BT


# speedup vs baseline: 1.1978x; 1.1467x over previous
import functools

import jax
import jax.numpy as jnp
from jax import lax
from jax.experimental import pallas as pl
from jax.experimental.pallas import tpu as pltpu

_NEG = -1e30


def _round_up(x, m):
    return ((x + m - 1) // m) * m


def _gather_ce_kernel(idx_sref, tgt_ref, emb_ref, logits_ref, part_ref,
                      emb3_scr, copy_sem,
                      *, tile, groups_per_trip, v_pad, bt):
    i = pl.program_id(0)
    base = i * tile

    @pl.when(i == 0)
    def _():
        cp = pltpu.make_async_copy(emb_ref, emb3_scr.at[:, 0, :], copy_sem)
        cp.start()
        cp.wait()

    def trip(o, carry):
        for g in range(groups_per_trip):
            row0 = o * groups_per_trip * 8 + g * 8
            rows = [emb3_scr[idx_sref[base + row0 + k], 0, :]
                    for k in range(8)]
            x8 = jnp.stack(rows, axis=0)
            logits_ref[pl.ds(pl.multiple_of(row0, 8), 8), :] = x8
        return carry

    lax.fori_loop(0, tile // (8 * groups_per_trip), trip, 0)

    x = logits_ref[...]
    m = jnp.max(x, axis=-1, keepdims=True)
    s = jnp.sum(jnp.exp(x - m), axis=-1, keepdims=True)
    lane = lax.broadcasted_iota(jnp.int32, x.shape, 1)
    tgt_logit = jnp.sum(jnp.where(lane == tgt_ref[...], x, 0.0),
                        axis=-1, keepdims=True)
    per_ex = m + jnp.log(s) - tgt_logit
    row_id = base + lax.broadcasted_iota(jnp.int32, (tile, 1), 0)
    per_ex = jnp.where(row_id < bt, per_ex, 0.0)
    part_ref[0, 0, :] = jnp.sum(per_ex).reshape(1)


def _pad_1d(tok, bt, bt_pad):
    tok = tok.reshape(bt).astype(jnp.int32)
    if bt_pad != bt:
        tok = jnp.concatenate([tok, jnp.zeros((bt_pad - bt,), jnp.int32)])
    return tok


def kernel(emb_table, idx, targets, *, tile=1024, groups_per_trip=4):
    B, T = idx.shape
    V = emb_table.shape[0]
    BT = B * T

    v_pad = _round_up(V, 128)
    tile = min(tile, _round_up(BT, 8))
    bt_pad = _round_up(BT, tile)
    num_tiles = bt_pad // tile

    if v_pad == V:
        emb_pad = emb_table.astype(jnp.float32)
    else:
        emb_pad = jnp.full((v_pad, v_pad), _NEG, dtype=jnp.float32)
        emb_pad = emb_pad.at[:V, :V].set(emb_table.astype(jnp.float32))

    idx_flat = _pad_1d(idx, BT, bt_pad)
    tgt_flat = _pad_1d(targets if targets is not None else idx, BT, bt_pad)
    tgt2 = tgt_flat.reshape(bt_pad, 1)

    body = functools.partial(_gather_ce_kernel, tile=tile,
                             groups_per_trip=groups_per_trip,
                             v_pad=v_pad, bt=BT)

    logits, partials = pl.pallas_call(
        body,
        out_shape=(
            jax.ShapeDtypeStruct((bt_pad, v_pad), jnp.float32),
            jax.ShapeDtypeStruct((num_tiles, 1, 1), jnp.float32),
        ),
        grid=(num_tiles,),
        in_specs=[
            pl.BlockSpec(memory_space=pltpu.SMEM),
            pl.BlockSpec((tile, 1), lambda i: (i, 0)),
            pl.BlockSpec(memory_space=pl.ANY),
        ],
        out_specs=(
            pl.BlockSpec((tile, v_pad), lambda i: (i, 0)),
            pl.BlockSpec((1, 1, 1), lambda i: (i, 0, 0)),
        ),
        scratch_shapes=[
            pltpu.VMEM((v_pad, 1, v_pad), jnp.float32),
            pltpu.SemaphoreType.DMA,
        ],
        compiler_params=pltpu.CompilerParams(
            dimension_semantics=("arbitrary",),
            vmem_limit_bytes=58 * 1024 * 1024,
        ),
        cost_estimate=pl.CostEstimate(
            flops=8 * bt_pad * v_pad,
            transcendentals=bt_pad * v_pad,
            bytes_accessed=(v_pad * v_pad * 4 + bt_pad * v_pad * 4
                            + 2 * bt_pad * 4),
        ),
    )(idx_flat, tgt2, emb_pad)

    if bt_pad != BT or v_pad != V:
        logits = logits[:BT, :V]

    if targets is None:
        return logits.reshape(B, T, V), None

    return logits, jnp.sum(partials) / BT
